# Initial kernel scaffold; baseline (speedup 1.0000x reference)
import jax
import jax.numpy as jnp
from jax.experimental import pallas as pl


def kernel(ego, neb, neb_confidence_map, neb_point_cloud_range, edge_index, edge_vals, params):
    raise NotImplementedError("write your pallas kernel here")


# speedup vs baseline: 1.0558x; 1.0558x over previous
import math
import jax
import jax.numpy as jnp
from jax.experimental import pallas as pl

C = 128
H = 128
W = 128
E_RAW = 131072
NHALF = H * W
N = 2 * H * W
NUM_HEADS = 8
DH = C // NUM_HEADS
N_LAYERS = 2
SEARCH_RANGE = 3.0
PCR = (-140.8, -40.0, -3.0, 140.8, 40.0, 1.0)


def _ln(x, g, b):
    m = x.mean(-1, keepdims=True)
    v = ((x - m) ** 2).mean(-1, keepdims=True)
    return (x - m) / jnp.sqrt(v + 1e-5) * g + b


def _dis_encode(x, scale, Wd, bd):
    i = jnp.arange(H, dtype=jnp.float32)[:, None]
    j = jnp.arange(W, dtype=jnp.float32)[None, :]
    d = jnp.sqrt(jnp.square(scale[0] * (i - (H - 1) / 2.0)) + jnp.square(scale[1] * (j - (W - 1) / 2.0)))
    c = jnp.arange(C)
    ce = ((c // 2) * 2).astype(jnp.float32)
    div = jnp.exp(-ce * (math.log(10000.0) / C))
    arg = d[None, :, :] * div[:, None, None]
    pe = jnp.where((c % 2 == 0)[:, None, None], jnp.sin(arg), jnp.cos(arg)) / math.sqrt(C)
    return x + jnp.einsum('oc,chw->ohw', Wd, pe) + bd[:, None, None]


def _gt_layer(h, e, src, dst, p):
    q = (h @ p['Wq']).reshape(-1, NUM_HEADS, DH)
    k = (h @ p['Wk']).reshape(-1, NUM_HEADS, DH)
    v = (h @ p['Wv']).reshape(-1, NUM_HEADS, DH)
    pe = (e @ p['We']).reshape(-1, NUM_HEADS, DH)
    score = (k[src] * q[dst]) / math.sqrt(DH) * pe
    e_out = score.reshape(-1, C)
    logits = jnp.clip(score.sum(-1), -5.0, 5.0)
    ex = jnp.exp(logits)
    den = jax.ops.segment_sum(ex, dst, num_segments=N) + 1e-9
    hagg = jax.ops.segment_sum(ex[:, :, None] * v[src], dst, num_segments=N)
    hagg = (hagg / den[:, :, None]).reshape(-1, C)
    h1 = _ln(h + hagg @ p['Wo'] + p['bo'], p['ln1g'], p['ln1b'])
    e1 = _ln(e + e_out @ p['WoE'] + p['boE'], p['lnE1g'], p['lnE1b'])
    h2 = _ln(h1 + jax.nn.relu(h1 @ p['Wf1'] + p['bf1']) @ p['Wf2'] + p['bf2'], p['ln2g'], p['ln2b'])
    e2 = _ln(e1 + jax.nn.relu(e1 @ p['WfE1'] + p['bfE1']) @ p['WfE2'] + p['bfE2'], p['lnE2g'], p['lnE2b'])
    return h2, e2


def _out_transpose_body(h_ref, o_ref):
    o_ref[...] = h_ref[...].reshape(8, W, C).transpose(2, 0, 1)


def kernel(ego, neb, neb_confidence_map, neb_point_cloud_range, edge_index, edge_vals, params):
    neb_pcr = neb_point_cloud_range
    sc_ego = jnp.array([(PCR[4] - PCR[1]) / H, (PCR[3] - PCR[0]) / W], dtype=jnp.float32)
    sc_neb = jnp.stack([(neb_pcr[4] - neb_pcr[1]) / H, (neb_pcr[3] - neb_pcr[0]) / W])
    src = edge_index[0].astype(jnp.int32) + NHALF
    dst = edge_index[1].astype(jnp.int32)
    dis = edge_vals[:, 0]
    delta = ((dis[:, None] @ params['Wd1'] + params['bd1']) @ params['Wd2'] + params['bd2'])[:, 0]
    ddd = delta / (dis + 1e-7)
    v0 = (edge_vals[:, 0] + delta) / SEARCH_RANGE
    ego_area = sc_ego[0] * sc_ego[1]
    neb_area = sc_neb[0] * sc_neb[1]
    ddn = delta ** 2 / neb_area
    v1 = (edge_vals[:, 1] + ddn) / (1.0 + ddn)
    ddn2 = delta ** 2 / ego_area
    v1n = (edge_vals[:, 1] * neb_area / ego_area + ddn2) / (1.0 + ddn2)
    v2 = (edge_vals[:, 2] + ddd) / (1.0 + ddd)
    v3 = (edge_vals[:, 3] + ddd) / (1.0 + ddd)
    vals = jnp.concatenate([jnp.stack([v0, v1, v2, v3], -1), jnp.stack([v0, v1n, v2, -v3], -1)], 0)
    loop = jnp.arange(N, dtype=jnp.int32)
    s_all = jnp.concatenate([src, dst, loop])
    d_all = jnp.concatenate([dst, src, loop])
    vals = jnp.concatenate([vals, jnp.tile(jnp.array([[0.0, 1.0, 0.0, 1.0]], dtype=jnp.float32), (N, 1))], 0)
    ego_p = _dis_encode(ego, sc_ego, params['W_dis'], params['b_dis'])
    neb_p = _dis_encode(neb, sc_neb, params['W_dis'], params['b_dis'])
    h = jnp.concatenate([ego_p.reshape(C, -1).T, neb_p.reshape(C, -1).T], 0)
    e = vals @ params['W_ee'] + params['b_ee']
    for l in range(N_LAYERS):
        h, e = _gt_layer(h, e, s_all, d_all, params['layers'][l])

    hout = h[:NHALF]
    out = pl.pallas_call(
        _out_transpose_body,
        grid=(H // 8,),
        in_specs=[pl.BlockSpec((8 * W, C), lambda i: (i, 0))],
        out_specs=pl.BlockSpec((C, 8, W), lambda i: (0, i, 0)),
        out_shape=jax.ShapeDtypeStruct((C, H, W), jnp.float32),
    )(hout)
    return out


# speedup vs baseline: 22.2989x; 21.1210x over previous
import functools
import math

import jax
import jax.numpy as jnp
from jax import lax
from jax.experimental import pallas as pl
from jax.experimental.pallas import tpu as pltpu
from jax.experimental.pallas import tpu_sc as plsc

C = 128
H = 128
W = 128
E_RAW = 131072
NHALF = H * W
N = 2 * H * W
NUM_HEADS = 8
DH = C // NUM_HEADS
N_LAYERS = 2
SEARCH_RANGE = 3.0
PCR = (-140.8, -40.0, -3.0, 140.8, 40.0, 1.0)

NC = 2
NS = 16
NW = NC * NS
E_TOT = 2 * E_RAW + N

_SC_MESH = plsc.VectorSubcoreMesh(core_axis_name="c", subcore_axis_name="s")


def _make_gather3():
    per_w = E_TOT // NW
    CH = 128
    n_ch = per_w // CH

    @functools.partial(
        pl.kernel,
        mesh=_SC_MESH,
        out_type=[jax.ShapeDtypeStruct((E_TOT, C), jnp.float32)] * 3,
        scratch_types=[
            pltpu.VMEM((CH,), jnp.int32),
            pltpu.VMEM((CH,), jnp.int32),
            pltpu.VMEM((CH, C), jnp.float32),
            pltpu.VMEM((CH, C), jnp.float32),
            pltpu.VMEM((CH, C), jnp.float32),
            pltpu.SemaphoreType.DMA,
        ],
    )
    def gather3(k_hbm, q_hbm, v_hbm, src_hbm, dst_hbm,
                ok_hbm, oq_hbm, ov_hbm, idxs, idxd, rk, rq, rv, sem):
        wid = lax.axis_index("s") * NC + lax.axis_index("c")
        base = wid * per_w

        @pl.loop(0, n_ch)
        def _(i):
            off = base + i * CH
            pltpu.sync_copy(src_hbm.at[pl.ds(off, CH)], idxs)
            pltpu.sync_copy(dst_hbm.at[pl.ds(off, CH)], idxd)
            ck = pltpu.async_copy(k_hbm.at[idxs], rk, sem)
            cq = pltpu.async_copy(q_hbm.at[idxd], rq, sem)
            cv = pltpu.async_copy(v_hbm.at[idxs], rv, sem)
            ck.wait()
            cq.wait()
            cv.wait()
            pltpu.sync_copy(rk, ok_hbm.at[pl.ds(off, CH)])
            pltpu.sync_copy(rq, oq_hbm.at[pl.ds(off, CH)])
            pltpu.sync_copy(rv, ov_hbm.at[pl.ds(off, CH)])

    return gather3


_GATHER3 = _make_gather3()


_QR = 8192
_ACC_R = _QR + 16
_CH = 128


def _make_scatter2():
    n1 = E_RAW // NS // _CH
    n2 = NHALF // NS // _CH
    ZSTR = _ACC_R // NS
    OSTR = _QR // NS

    @functools.partial(
        pl.kernel,
        mesh=_SC_MESH,
        out_type=[jax.ShapeDtypeStruct((N, C), jnp.float32)] * 2,
        scratch_types=[
            pltpu.VMEM_SHARED((_ACC_R, C), jnp.float32),
            pltpu.VMEM((_CH,), jnp.int32),
            pltpu.VMEM((_CH,), jnp.int32),
            pltpu.VMEM((_CH, C), jnp.float32),
        ],
    )
    def scatter(conA_hbm, conB_hbm, dstl_hbm, z_hbm, outA_hbm, outB_hbm,
                acc, idxr, idx, buf):
        c = lax.axis_index("c")
        s = lax.axis_index("s")

        def clamp_chunk(q):
            for t in range(_CH // 16):
                v = idxr[pl.ds(t * 16, 16)]
                lo = v - q * _QR
                ok = jnp.logical_and(lo >= 0, lo < _QR)
                idx[pl.ds(t * 16, 16)] = jnp.where(ok, lo, _QR)

        def one_pass(con_hbm, out_hbm, q):
            pltpu.sync_copy(z_hbm, acc.at[pl.ds(s * ZSTR, ZSTR)])
            plsc.subcore_barrier()

            base1 = c * E_RAW + s * (E_RAW // NS)
            @pl.loop(0, n1)
            def _(i):
                off = base1 + i * _CH
                pltpu.sync_copy(dstl_hbm.at[pl.ds(off, _CH)], idxr)
                pltpu.sync_copy(con_hbm.at[pl.ds(off, _CH)], buf)
                clamp_chunk(q)
                pltpu.sync_copy(buf, acc.at[idx], add=True)

            base2 = 2 * E_RAW + c * NHALF + s * (NHALF // NS)
            @pl.loop(0, n2)
            def _(i):
                off = base2 + i * _CH
                pltpu.sync_copy(dstl_hbm.at[pl.ds(off, _CH)], idxr)
                pltpu.sync_copy(con_hbm.at[pl.ds(off, _CH)], buf)
                clamp_chunk(q)
                pltpu.sync_copy(buf, acc.at[idx], add=True)

            plsc.subcore_barrier()
            pltpu.sync_copy(
                acc.at[pl.ds(s * OSTR, OSTR)],
                out_hbm.at[pl.ds(c * NHALF + q * _QR + s * OSTR, OSTR)])
            plsc.subcore_barrier()

        one_pass(conA_hbm, outA_hbm, 0)
        one_pass(conA_hbm, outA_hbm, 1)
        one_pass(conB_hbm, outB_hbm, 0)
        one_pass(conB_hbm, outB_hbm, 1)

    return scatter


_SCATTER2 = _make_scatter2()


_BR = 512


def _escore_body(ks_ref, qd_ref, pe_ref, vs_ref, eout_ref, ca_ref, cb_ref):
    s = ks_ref[...] * qd_ref[...] * pe_ref[...] * (1.0 / math.sqrt(DH))
    eout_ref[...] = s
    ch = lax.broadcasted_iota(jnp.int32, (C, NUM_HEADS), 0) // DH
    hh = lax.broadcasted_iota(jnp.int32, (C, NUM_HEADS), 1)
    hm = (ch == hh).astype(jnp.float32)
    logits = jnp.clip(jax.lax.dot(s, hm, precision=jax.lax.Precision.HIGHEST),
                      -5.0, 5.0)
    ex = jnp.exp(logits)
    exf = jax.lax.dot(ex, hm.T, precision=jax.lax.Precision.HIGHEST)
    exv = exf * vs_ref[...]
    ca_ref[...] = jnp.concatenate(
        [exv[:, :C // 2], ex, jnp.zeros((_BR, 56), jnp.float32)], axis=1)
    cb_ref[...] = jnp.concatenate(
        [exv[:, C // 2:], jnp.zeros((_BR, C // 2), jnp.float32)], axis=1)


def _escore(ksrc, qdst, pe, vsrc):
    grid = (E_TOT // _BR,)
    return pl.pallas_call(
        _escore_body,
        grid=grid,
        in_specs=[pl.BlockSpec((_BR, C), lambda i: (i, 0))] * 4,
        out_specs=[pl.BlockSpec((_BR, C), lambda i: (i, 0))] * 3,
        out_shape=[jax.ShapeDtypeStruct((E_TOT, C), jnp.float32)] * 3,
    )(ksrc, qdst, pe, vsrc)


def _ln(x, g, b):
    m = x.mean(-1, keepdims=True)
    v = ((x - m) ** 2).mean(-1, keepdims=True)
    return (x - m) / jnp.sqrt(v + 1e-5) * g + b


def _dis_encode(x, scale, Wd, bd):
    i = jnp.arange(H, dtype=jnp.float32)[:, None]
    j = jnp.arange(W, dtype=jnp.float32)[None, :]
    d = jnp.sqrt(jnp.square(scale[0] * (i - (H - 1) / 2.0)) + jnp.square(scale[1] * (j - (W - 1) / 2.0)))
    c = jnp.arange(C)
    ce = ((c // 2) * 2).astype(jnp.float32)
    div = jnp.exp(-ce * (math.log(10000.0) / C))
    arg = d[None, :, :] * div[:, None, None]
    pe = jnp.where((c % 2 == 0)[:, None, None], jnp.sin(arg), jnp.cos(arg)) / math.sqrt(C)
    return x + jnp.einsum('oc,chw->ohw', Wd, pe) + bd[:, None, None]


def _gt_layer(h, e, src_g, dst_g, dst_l, zrows, p):
    q = h @ p['Wq']
    k = h @ p['Wk']
    v = h @ p['Wv']
    pe = e @ p['We']
    ksrc, qdst, vsrc = _GATHER3(k, q, v, src_g, dst_g)
    e_out, ca, cb = _escore(ksrc, qdst, pe, vsrc)
    aggA, aggB = _SCATTER2(ca, cb, dst_l, zrows)
    den = aggA[:, 64:72] + 1e-9
    hagg = jnp.concatenate([aggA[:, :64], aggB[:, :64]], axis=1)
    hagg = hagg / jnp.repeat(den, DH, axis=1)
    h1 = _ln(h + hagg @ p['Wo'] + p['bo'], p['ln1g'], p['ln1b'])
    e1 = _ln(e + e_out @ p['WoE'] + p['boE'], p['lnE1g'], p['lnE1b'])
    h2 = _ln(h1 + jax.nn.relu(h1 @ p['Wf1'] + p['bf1']) @ p['Wf2'] + p['bf2'], p['ln2g'], p['ln2b'])
    e2 = _ln(e1 + jax.nn.relu(e1 @ p['WfE1'] + p['bfE1']) @ p['WfE2'] + p['bfE2'], p['lnE2g'], p['lnE2b'])
    return h2, e2


def _out_transpose_body(h_ref, o_ref):
    o_ref[...] = h_ref[...].reshape(8, W, C).transpose(2, 0, 1)


def kernel(ego, neb, neb_confidence_map, neb_point_cloud_range, edge_index, edge_vals, params):
    neb_pcr = neb_point_cloud_range
    sc_ego = jnp.array([(PCR[4] - PCR[1]) / H, (PCR[3] - PCR[0]) / W], dtype=jnp.float32)
    sc_neb = jnp.stack([(neb_pcr[4] - neb_pcr[1]) / H, (neb_pcr[3] - neb_pcr[0]) / W])
    ei0 = edge_index[0].astype(jnp.int32)
    ei1 = edge_index[1].astype(jnp.int32)
    src = ei0 + NHALF
    dst = ei1
    dis = edge_vals[:, 0]
    delta = ((dis[:, None] @ params['Wd1'] + params['bd1']) @ params['Wd2'] + params['bd2'])[:, 0]
    ddd = delta / (dis + 1e-7)
    v0 = (edge_vals[:, 0] + delta) / SEARCH_RANGE
    ego_area = sc_ego[0] * sc_ego[1]
    neb_area = sc_neb[0] * sc_neb[1]
    ddn = delta ** 2 / neb_area
    v1 = (edge_vals[:, 1] + ddn) / (1.0 + ddn)
    ddn2 = delta ** 2 / ego_area
    v1n = (edge_vals[:, 1] * neb_area / ego_area + ddn2) / (1.0 + ddn2)
    v2 = (edge_vals[:, 2] + ddd) / (1.0 + ddd)
    v3 = (edge_vals[:, 3] + ddd) / (1.0 + ddd)
    vals = jnp.concatenate([jnp.stack([v0, v1, v2, v3], -1), jnp.stack([v0, v1n, v2, -v3], -1)], 0)
    loop = jnp.arange(N, dtype=jnp.int32)
    ar = jnp.arange(NHALF, dtype=jnp.int32)
    src_g = jnp.concatenate([src, dst, loop])
    dst_g = jnp.concatenate([dst, src, loop])
    dst_l = jnp.concatenate([ei1, ei0, ar, ar])
    vals = jnp.concatenate([vals, jnp.tile(jnp.array([[0.0, 1.0, 0.0, 1.0]], dtype=jnp.float32), (N, 1))], 0)
    zrows = jnp.zeros((_ACC_R // NS, C), jnp.float32)
    ego_p = _dis_encode(ego, sc_ego, params['W_dis'], params['b_dis'])
    neb_p = _dis_encode(neb, sc_neb, params['W_dis'], params['b_dis'])
    h = jnp.concatenate([ego_p.reshape(C, -1).T, neb_p.reshape(C, -1).T], 0)
    e = vals @ params['W_ee'] + params['b_ee']
    for l in range(N_LAYERS):
        h, e = _gt_layer(h, e, src_g, dst_g, dst_l, zrows, params['layers'][l])

    hout = h[:NHALF]
    out = pl.pallas_call(
        _out_transpose_body,
        grid=(H // 8,),
        in_specs=[pl.BlockSpec((8 * W, C), lambda i: (i, 0))],
        out_specs=pl.BlockSpec((C, 8, W), lambda i: (0, i, 0)),
        out_shape=jax.ShapeDtypeStruct((C, H, W), jnp.float32),
    )(hout)
    return out
